# baseline (speedup 1.0000x reference)
import functools

import jax
import jax.numpy as jnp
from jax.experimental import pallas as pl
from jax.experimental.pallas import tpu as pltpu

_P = 1000
_U = 125
_NP = 50000
_NC = 6250
_VTOT = 3420
_VPAD = 3456
_SIZES = (12, 42, 162, 642, 2562)
_OFFS = (0, 12, 54, 216, 858)
_VPADS = (128, 128, 256, 768, 2688)
_BLKS = (128, 128, 256, 768, 384)


def _nrm(v):
    n = jnp.linalg.norm(v, axis=-1, keepdims=True)
    return v / jnp.maximum(n, 1e-12)


def _ins(v, i, acc):
    v1, v2, v3, p1, p2, p3 = acc
    c1 = v > v1
    c2 = v > v2
    c3 = v > v3
    t1 = jnp.minimum(v, v1)
    n1 = jnp.maximum(v, v1)
    t2 = jnp.minimum(t1, v2)
    n2 = jnp.maximum(t1, v2)
    n3 = jnp.maximum(t2, v3)
    q1 = jnp.where(c1, i, p1)
    q2 = jnp.where(c1, p1, jnp.where(c2, i, p2))
    q3 = jnp.where(c2, p2, jnp.where(c3, i, p3))
    return n1, n2, n3, q1, q2, q3


def _top3_body(x_ref, it_ref, dirs_ref, o_ref, d_ref,
               v1_ref, v2_ref, v3_ref, p1_ref, p2_ref, p3_ref):
    pid = pl.program_id(0)
    last = pl.num_programs(0) - 1

    @pl.when(pid == 0)
    def _init():
        for r in (v1_ref, v2_ref, v3_ref):
            r[...] = jnp.full(r.shape, -3.0, jnp.float32)
        for r in (p1_ref, p2_ref, p3_ref):
            r[...] = jnp.zeros(r.shape, jnp.float32)

    d_ref[...] = jax.lax.dot_general(
        x_ref[...], dirs_ref[...], (((1,), (0,)), ((), ())),
        preferred_element_type=jnp.float32)

    def body(k, _):
        acc = (v1_ref[...], v2_ref[...], v3_ref[...],
               p1_ref[...], p2_ref[...], p3_ref[...])
        for u in range(_U):
            r = (k * _U + u) * 8
            slab = d_ref[pl.ds(r, 8), :]
            ii = it_ref[pl.ds(r, 8), 0:1]
            acc = _ins(slab, ii, acc)
        v1_ref[...], v2_ref[...], v3_ref[...] = acc[0], acc[1], acc[2]
        p1_ref[...], p2_ref[...], p3_ref[...] = acc[3], acc[4], acc[5]
        return 0

    jax.lax.fori_loop(0, _P // (8 * _U), body, 0)

    @pl.when(pid == last)
    def _fin():
        z1 = jnp.full((1, _VPAD), -3.0, jnp.float32)
        fin = (z1, z1, z1, jnp.zeros((1, _VPAD), jnp.float32),
               jnp.zeros((1, _VPAD), jnp.float32),
               jnp.zeros((1, _VPAD), jnp.float32))
        for s in range(8):
            for vr, pr in ((v1_ref, p1_ref), (v2_ref, p2_ref),
                           (v3_ref, p3_ref)):
                fin = _ins(vr[s:s + 1, :], pr[s:s + 1, :], fin)
        g1, g2, g3 = fin[3], fin[4], fin[5]
        inten = ((g1 + g2) + g3) / 3.0
        o_ref[...] = jnp.concatenate(
            [inten, jnp.zeros((7, _VPAD), jnp.float32)], axis=0)


def _top3_call(xp, itp, dirs_k8):
    return pl.pallas_call(
        _top3_body,
        grid=(_NP // _P,),
        in_specs=[
            pl.BlockSpec((_P, 8), lambda i: (i, 0)),
            pl.BlockSpec((_P, 8), lambda i: (i, 0)),
            pl.BlockSpec((8, _VPAD), lambda i: (0, 0)),
        ],
        out_specs=pl.BlockSpec((8, _VPAD), lambda i: (0, 0)),
        out_shape=jax.ShapeDtypeStruct((8, _VPAD), jnp.float32),
        scratch_shapes=[pltpu.VMEM((_P, _VPAD), jnp.float32)] +
                       [pltpu.VMEM((8, _VPAD), jnp.float32)] * 6,
    )(xp, itp, dirs_k8)


def _knn_rows(p, pt, v_real, vp):
    d0 = p[:, 0:1] - pt[0:1, :]
    d1 = p[:, 1:2] - pt[1:2, :]
    d2 = p[:, 2:3] - pt[2:3, :]
    dd = jnp.sqrt((d0 * d0 + d1 * d1) + d2 * d2)
    b = p.shape[0]
    col = jax.lax.broadcasted_iota(jnp.int32, (b, vp), 1)
    dd = jnp.where(col >= v_real, jnp.float32(3e9), dd)
    outs = []
    for _ in range(7):
        m = jnp.min(dd, axis=1, keepdims=True)
        ridx = jnp.min(jnp.where(dd == m, col, vp), axis=1, keepdims=True)
        outs.append(ridx)
        dd = jnp.where(col == ridx, jnp.float32(4e9), dd)
    return jnp.concatenate(outs + [outs[-1]], axis=1)


def _knn_body(p_ref, pt_ref, o_ref, *, v_real, vp):
    o_ref[...] = _knn_rows(p_ref[...], pt_ref[...], v_real, vp)


def _knn_call(p8, pT, v_real, vp, blk):
    body = functools.partial(_knn_body, v_real=v_real, vp=vp)
    return pl.pallas_call(
        body,
        grid=(vp // blk,),
        in_specs=[
            pl.BlockSpec((blk, 8), lambda i: (i, 0)),
            pl.BlockSpec((8, vp), lambda i: (0, 0)),
        ],
        out_specs=pl.BlockSpec((blk, 8), lambda i: (i, 0)),
        out_shape=jax.ShapeDtypeStruct((vp, 8), jnp.int32),
    )(p8, pT)


def _knn_small_body(p_ref, pt_ref, o_ref):
    lvl = pl.program_id(0)
    vr = jnp.where(lvl == 0, 12,
                   jnp.where(lvl == 1, 42, jnp.where(lvl == 2, 162, 642)))
    o_ref[...] = _knn_rows(p_ref[...], pt_ref[0], vr, 768)


def _knn_small_call(p8cat, ptcat):
    return pl.pallas_call(
        _knn_small_body,
        grid=(4,),
        in_specs=[
            pl.BlockSpec((768, 8), lambda i: (i, 0)),
            pl.BlockSpec((1, 8, 768), lambda i: (i, 0, 0)),
        ],
        out_specs=pl.BlockSpec((768, 8), lambda i: (i, 0)),
        out_shape=jax.ShapeDtypeStruct((3072, 8), jnp.int32),
    )(p8cat, ptcat)


def kernel(points_xyz, intensity, neighbor_k, verts0, verts1, verts2,
           verts3, verts4):
    del neighbor_k
    vlist = (verts0, verts1, verts2, verts3, verts4)

    com = jnp.mean(points_xyz, axis=0, keepdims=True)
    xn = _nrm(points_xyz - com)
    dirs_cat = jnp.concatenate([_nrm(v) for v in vlist], axis=0)
    dirs_k8 = jnp.pad(dirs_cat, ((0, _VPAD - _VTOT), (0, 5))).T
    xp = jnp.pad(xn, ((0, 0), (0, 5))).reshape(8, _NC, 8)
    xp = xp.transpose(1, 0, 2).reshape(_NP, 8)
    itp = jnp.pad(intensity, ((0, 0), (0, 7))).reshape(8, _NC, 8)
    itp = itp.transpose(1, 0, 2).reshape(_NP, 8)
    inten_all = _top3_call(xp, itp, dirs_k8)[0, :]

    p8cat = jnp.concatenate(
        [jnp.pad(v, ((0, 768 - n), (0, 5)))
         for v, n in zip(vlist[:4], _SIZES[:4])], axis=0)
    ptcat = jnp.stack(
        [jnp.pad(v.T, ((0, 5), (0, 768 - n)))
         for v, n in zip(vlist[:4], _SIZES[:4])], axis=0)
    small7 = _knn_small_call(p8cat, ptcat)

    out = []
    for lvl, (v_real, off, pos) in enumerate(zip(_SIZES, _OFFS, vlist)):
        if lvl < 4:
            nbr7 = small7[lvl * 768: lvl * 768 + v_real]
        else:
            p8 = jnp.pad(pos, ((0, 2688 - v_real), (0, 5)))
            pT = jnp.pad(pos.T, ((0, 5), (0, 2688 - v_real)))
            nbr7 = _knn_call(p8, pT, v_real, 2688, 384)
        nbr = nbr7[:v_real, 1:7]
        inten_l = inten_all[off:off + v_real].reshape(v_real, 1)
        feats = jnp.concatenate([inten_l, pos], axis=1)
        src = jnp.broadcast_to(
            jnp.arange(v_real, dtype=jnp.int32)[:, None], (v_real, 6))
        e = jnp.stack([src.reshape(-1), nbr.reshape(-1)], axis=0)
        e = jnp.concatenate([e, e[::-1, :]], axis=1)
        out.extend([pos, feats, e])
    return tuple(out)


# speedup vs baseline: 1.0005x; 1.0005x over previous
import functools

import jax
import jax.numpy as jnp
from jax.experimental import pallas as pl
from jax.experimental.pallas import tpu as pltpu

_P = 2000
_U = 250
_NP = 50000
_NC = 6250
_VTOT = 3420
_VPAD = 3456
_SIZES = (12, 42, 162, 642, 2562)
_OFFS = (0, 12, 54, 216, 858)
_VPADS = (128, 128, 256, 768, 2688)
_BLKS = (128, 128, 256, 768, 384)


def _nrm(v):
    n = jnp.linalg.norm(v, axis=-1, keepdims=True)
    return v / jnp.maximum(n, 1e-12)


def _ins(v, i, acc):
    v1, v2, v3, p1, p2, p3 = acc
    c1 = v > v1
    c2 = v > v2
    c3 = v > v3
    n1 = jnp.where(c1, v, v1)
    n2 = jnp.where(c1, v1, jnp.where(c2, v, v2))
    n3 = jnp.where(c2, v2, jnp.where(c3, v, v3))
    q1 = jnp.where(c1, i, p1)
    q2 = jnp.where(c1, p1, jnp.where(c2, i, p2))
    q3 = jnp.where(c2, p2, jnp.where(c3, i, p3))
    return n1, n2, n3, q1, q2, q3


def _top3_body(x_ref, it_ref, dirs_ref, o_ref, d_ref,
               v1_ref, v2_ref, v3_ref, p1_ref, p2_ref, p3_ref):
    pid = pl.program_id(0)
    last = pl.num_programs(0) - 1

    @pl.when(pid == 0)
    def _init():
        for r in (v1_ref, v2_ref, v3_ref):
            r[...] = jnp.full(r.shape, -3.0, jnp.float32)
        for r in (p1_ref, p2_ref, p3_ref):
            r[...] = jnp.zeros(r.shape, jnp.float32)

    d_ref[...] = jax.lax.dot_general(
        x_ref[...], dirs_ref[...], (((1,), (0,)), ((), ())),
        preferred_element_type=jnp.float32)

    def body(k, _):
        acc = (v1_ref[...], v2_ref[...], v3_ref[...],
               p1_ref[...], p2_ref[...], p3_ref[...])
        for u in range(_U):
            r = (k * _U + u) * 8
            slab = d_ref[pl.ds(r, 8), :]
            ii = it_ref[pl.ds(r, 8), 0:1]
            acc = _ins(slab, ii, acc)
        v1_ref[...], v2_ref[...], v3_ref[...] = acc[0], acc[1], acc[2]
        p1_ref[...], p2_ref[...], p3_ref[...] = acc[3], acc[4], acc[5]
        return 0

    jax.lax.fori_loop(0, _P // (8 * _U), body, 0)

    @pl.when(pid == last)
    def _fin():
        z1 = jnp.full((1, _VPAD), -3.0, jnp.float32)
        fin = (z1, z1, z1, jnp.zeros((1, _VPAD), jnp.float32),
               jnp.zeros((1, _VPAD), jnp.float32),
               jnp.zeros((1, _VPAD), jnp.float32))
        for s in range(8):
            for vr, pr in ((v1_ref, p1_ref), (v2_ref, p2_ref),
                           (v3_ref, p3_ref)):
                fin = _ins(vr[s:s + 1, :], pr[s:s + 1, :], fin)
        g1, g2, g3 = fin[3], fin[4], fin[5]
        inten = ((g1 + g2) + g3) / 3.0
        o_ref[...] = jnp.concatenate(
            [inten, jnp.zeros((7, _VPAD), jnp.float32)], axis=0)


def _top3_call(xp, itp, dirs_k8):
    return pl.pallas_call(
        _top3_body,
        grid=(_NP // _P,),
        in_specs=[
            pl.BlockSpec((_P, 8), lambda i: (i, 0)),
            pl.BlockSpec((_P, 8), lambda i: (i, 0)),
            pl.BlockSpec((8, _VPAD), lambda i: (0, 0)),
        ],
        out_specs=pl.BlockSpec((8, _VPAD), lambda i: (0, 0)),
        out_shape=jax.ShapeDtypeStruct((8, _VPAD), jnp.float32),
        scratch_shapes=[pltpu.VMEM((_P, _VPAD), jnp.float32)] +
                       [pltpu.VMEM((8, _VPAD), jnp.float32)] * 6,
    )(xp, itp, dirs_k8)


def _knn_rows(p, pt, v_real, vp):
    d0 = p[:, 0:1] - pt[0:1, :]
    d1 = p[:, 1:2] - pt[1:2, :]
    d2 = p[:, 2:3] - pt[2:3, :]
    dd = jnp.sqrt((d0 * d0 + d1 * d1) + d2 * d2)
    b = p.shape[0]
    col = jax.lax.broadcasted_iota(jnp.int32, (b, vp), 1)
    dd = jnp.where(col >= v_real, jnp.float32(3e9), dd)
    outs = []
    for _ in range(7):
        m = jnp.min(dd, axis=1, keepdims=True)
        ridx = jnp.min(jnp.where(dd == m, col, vp), axis=1, keepdims=True)
        outs.append(ridx)
        dd = jnp.where(col == ridx, jnp.float32(4e9), dd)
    return jnp.concatenate(outs + [outs[-1]], axis=1)


def _knn_body(p_ref, pt_ref, o_ref, *, v_real, vp):
    o_ref[...] = _knn_rows(p_ref[...], pt_ref[...], v_real, vp)


def _knn_call(p8, pT, v_real, vp, blk):
    body = functools.partial(_knn_body, v_real=v_real, vp=vp)
    return pl.pallas_call(
        body,
        grid=(vp // blk,),
        in_specs=[
            pl.BlockSpec((blk, 8), lambda i: (i, 0)),
            pl.BlockSpec((8, vp), lambda i: (0, 0)),
        ],
        out_specs=pl.BlockSpec((blk, 8), lambda i: (i, 0)),
        out_shape=jax.ShapeDtypeStruct((vp, 8), jnp.int32),
    )(p8, pT)


def _knn_small_body(p_ref, pt_ref, o_ref):
    lvl = pl.program_id(0)
    vr = jnp.where(lvl == 0, 12,
                   jnp.where(lvl == 1, 42, jnp.where(lvl == 2, 162, 642)))
    o_ref[...] = _knn_rows(p_ref[...], pt_ref[0], vr, 768)


def _knn_small_call(p8cat, ptcat):
    return pl.pallas_call(
        _knn_small_body,
        grid=(4,),
        in_specs=[
            pl.BlockSpec((768, 8), lambda i: (i, 0)),
            pl.BlockSpec((1, 8, 768), lambda i: (i, 0, 0)),
        ],
        out_specs=pl.BlockSpec((768, 8), lambda i: (i, 0)),
        out_shape=jax.ShapeDtypeStruct((3072, 8), jnp.int32),
    )(p8cat, ptcat)


def kernel(points_xyz, intensity, neighbor_k, verts0, verts1, verts2,
           verts3, verts4):
    del neighbor_k
    vlist = (verts0, verts1, verts2, verts3, verts4)

    com = jnp.mean(points_xyz, axis=0, keepdims=True)
    xn = _nrm(points_xyz - com)
    dirs_cat = jnp.concatenate([_nrm(v) for v in vlist], axis=0)
    dirs_k8 = jnp.pad(dirs_cat, ((0, _VPAD - _VTOT), (0, 5))).T
    xp = jnp.pad(xn, ((0, 0), (0, 5))).reshape(8, _NC, 8)
    xp = xp.transpose(1, 0, 2).reshape(_NP, 8)
    itp = jnp.pad(intensity, ((0, 0), (0, 7))).reshape(8, _NC, 8)
    itp = itp.transpose(1, 0, 2).reshape(_NP, 8)
    inten_all = _top3_call(xp, itp, dirs_k8)[0, :]

    p8cat = jnp.concatenate(
        [jnp.pad(v, ((0, 768 - n), (0, 5)))
         for v, n in zip(vlist[:4], _SIZES[:4])], axis=0)
    ptcat = jnp.stack(
        [jnp.pad(v.T, ((0, 5), (0, 768 - n)))
         for v, n in zip(vlist[:4], _SIZES[:4])], axis=0)
    small7 = _knn_small_call(p8cat, ptcat)

    out = []
    for lvl, (v_real, off, pos) in enumerate(zip(_SIZES, _OFFS, vlist)):
        if lvl < 4:
            nbr7 = small7[lvl * 768: lvl * 768 + v_real]
        else:
            p8 = jnp.pad(pos, ((0, 2688 - v_real), (0, 5)))
            pT = jnp.pad(pos.T, ((0, 5), (0, 2688 - v_real)))
            nbr7 = _knn_call(p8, pT, v_real, 2688, 384)
        nbr = nbr7[:v_real, 1:7]
        inten_l = inten_all[off:off + v_real].reshape(v_real, 1)
        feats = jnp.concatenate([inten_l, pos], axis=1)
        src = jnp.broadcast_to(
            jnp.arange(v_real, dtype=jnp.int32)[:, None], (v_real, 6))
        e = jnp.stack([src.reshape(-1), nbr.reshape(-1)], axis=0)
        e = jnp.concatenate([e, e[::-1, :]], axis=1)
        out.extend([pos, feats, e])
    return tuple(out)


# speedup vs baseline: 1.0048x; 1.0044x over previous
import functools

import jax
import jax.numpy as jnp
from jax.experimental import pallas as pl
from jax.experimental.pallas import tpu as pltpu

_P = 1000
_U = 125
_NP = 50000
_NC = 6250
_VTOT = 3420
_VPAD = 3456
_SIZES = (12, 42, 162, 642, 2562)
_OFFS = (0, 12, 54, 216, 858)


def _nrm(v):
    n = jnp.linalg.norm(v, axis=-1, keepdims=True)
    return v / jnp.maximum(n, 1e-12)


def _ins(v, i, acc):
    v1, v2, v3, p1, p2, p3 = acc
    c1 = v > v1
    c2 = v > v2
    c3 = v > v3
    n1 = jnp.where(c1, v, v1)
    n2 = jnp.where(c1, v1, jnp.where(c2, v, v2))
    n3 = jnp.where(c2, v2, jnp.where(c3, v, v3))
    q1 = jnp.where(c1, i, p1)
    q2 = jnp.where(c1, p1, jnp.where(c2, i, p2))
    q3 = jnp.where(c2, p2, jnp.where(c3, i, p3))
    return n1, n2, n3, q1, q2, q3


def _top3_body(x_ref, it_ref, dirs_ref, o_ref, d_ref,
               v1_ref, v2_ref, v3_ref, p1_ref, p2_ref, p3_ref):
    pid = pl.program_id(0)
    last = pl.num_programs(0) - 1

    @pl.when(pid == 0)
    def _init():
        for r in (v1_ref, v2_ref, v3_ref):
            r[...] = jnp.full(r.shape, -3.0, jnp.float32)
        for r in (p1_ref, p2_ref, p3_ref):
            r[...] = jnp.zeros(r.shape, jnp.float32)

    d_ref[...] = jax.lax.dot_general(
        x_ref[...], dirs_ref[...], (((1,), (0,)), ((), ())),
        preferred_element_type=jnp.float32)

    def body(k, _):
        acc = (v1_ref[...], v2_ref[...], v3_ref[...],
               p1_ref[...], p2_ref[...], p3_ref[...])
        for u in range(_U):
            r = (k * _U + u) * 8
            slab = d_ref[pl.ds(r, 8), :]
            ii = it_ref[pl.ds(r, 8), 0:1]
            acc = _ins(slab, ii, acc)
        v1_ref[...], v2_ref[...], v3_ref[...] = acc[0], acc[1], acc[2]
        p1_ref[...], p2_ref[...], p3_ref[...] = acc[3], acc[4], acc[5]
        return 0

    jax.lax.fori_loop(0, _P // (8 * _U), body, 0)

    @pl.when(pid == last)
    def _fin():
        z1 = jnp.full((1, _VPAD), -3.0, jnp.float32)
        fin = (z1, z1, z1, jnp.zeros((1, _VPAD), jnp.float32),
               jnp.zeros((1, _VPAD), jnp.float32),
               jnp.zeros((1, _VPAD), jnp.float32))
        for s in range(8):
            for vr, pr in ((v1_ref, p1_ref), (v2_ref, p2_ref),
                           (v3_ref, p3_ref)):
                fin = _ins(vr[s:s + 1, :], pr[s:s + 1, :], fin)
        g1, g2, g3 = fin[3], fin[4], fin[5]
        inten = ((g1 + g2) + g3) / 3.0
        o_ref[...] = jnp.concatenate(
            [inten, jnp.zeros((7, _VPAD), jnp.float32)], axis=0)


def _top3_call(xp, itp, dirs_k8):
    return pl.pallas_call(
        _top3_body,
        grid=(_NP // _P,),
        in_specs=[
            pl.BlockSpec((_P, 8), lambda i: (i, 0)),
            pl.BlockSpec((_P, 8), lambda i: (i, 0)),
            pl.BlockSpec((8, _VPAD), lambda i: (0, 0)),
        ],
        out_specs=pl.BlockSpec((8, _VPAD), lambda i: (0, 0)),
        out_shape=jax.ShapeDtypeStruct((8, _VPAD), jnp.float32),
        scratch_shapes=[pltpu.VMEM((_P, _VPAD), jnp.float32)] +
                       [pltpu.VMEM((8, _VPAD), jnp.float32)] * 6,
    )(xp, itp, dirs_k8)


def _knn_rows(p, pt, v_real, vp):
    d0 = p[:, 0:1] - pt[0:1, :]
    d1 = p[:, 1:2] - pt[1:2, :]
    d2 = p[:, 2:3] - pt[2:3, :]
    dd = jnp.sqrt((d0 * d0 + d1 * d1) + d2 * d2)
    b = p.shape[0]
    col = jax.lax.broadcasted_iota(jnp.int32, (b, vp), 1)
    dd = jnp.where(col >= v_real, jnp.float32(3e9), dd)
    outs = []
    for _ in range(7):
        m = jnp.min(dd, axis=1, keepdims=True)
        ridx = jnp.min(jnp.where(dd == m, col, vp), axis=1, keepdims=True)
        outs.append(ridx)
        dd = jnp.where(col == ridx, jnp.float32(4e9), dd)
    return jnp.concatenate(outs + [outs[-1]], axis=1)


def _knn_body(p_ref, pt_ref, o_ref, *, v_real, vp):
    o_ref[...] = _knn_rows(p_ref[...], pt_ref[...], v_real, vp)


def _knn_call(p8, pT, v_real, vp, blk):
    body = functools.partial(_knn_body, v_real=v_real, vp=vp)
    return pl.pallas_call(
        body,
        grid=(vp // blk,),
        in_specs=[
            pl.BlockSpec((blk, 8), lambda i: (i, 0)),
            pl.BlockSpec((8, vp), lambda i: (0, 0)),
        ],
        out_specs=pl.BlockSpec((blk, 8), lambda i: (i, 0)),
        out_shape=jax.ShapeDtypeStruct((vp, 8), jnp.int32),
    )(p8, pT)


def _knn_small_body(p_ref, pt_ref, o_ref):
    lvl = pl.program_id(0)
    vr = jnp.where(lvl == 0, 12,
                   jnp.where(lvl == 1, 42, jnp.where(lvl == 2, 162, 642)))
    o_ref[...] = _knn_rows(p_ref[...], pt_ref[0], vr, 768)


def _knn_small_call(p8cat, ptcat):
    return pl.pallas_call(
        _knn_small_body,
        grid=(4,),
        in_specs=[
            pl.BlockSpec((768, 8), lambda i: (i, 0)),
            pl.BlockSpec((1, 8, 768), lambda i: (i, 0, 0)),
        ],
        out_specs=pl.BlockSpec((768, 8), lambda i: (i, 0)),
        out_shape=jax.ShapeDtypeStruct((3072, 8), jnp.int32),
    )(p8cat, ptcat)


def kernel(points_xyz, intensity, neighbor_k, verts0, verts1, verts2,
           verts3, verts4):
    del neighbor_k
    vlist = (verts0, verts1, verts2, verts3, verts4)

    com = jnp.mean(points_xyz, axis=0, keepdims=True)
    xn = _nrm(points_xyz - com)
    dirs_cat = jnp.concatenate([_nrm(v) for v in vlist], axis=0)
    dirs_k8 = jnp.pad(dirs_cat, ((0, _VPAD - _VTOT), (0, 5))).T
    xp = jnp.pad(xn, ((0, 0), (0, 5))).reshape(8, _NC, 8)
    xp = xp.transpose(1, 0, 2).reshape(_NP, 8)
    itp = jnp.pad(intensity, ((0, 0), (0, 7))).reshape(8, _NC, 8)
    itp = itp.transpose(1, 0, 2).reshape(_NP, 8)
    inten_all = _top3_call(xp, itp, dirs_k8)[0, :]

    p8cat = jnp.concatenate(
        [jnp.pad(v, ((0, 768 - n), (0, 5)))
         for v, n in zip(vlist[:4], _SIZES[:4])], axis=0)
    ptcat = jnp.stack(
        [jnp.pad(v.T, ((0, 5), (0, 768 - n)))
         for v, n in zip(vlist[:4], _SIZES[:4])], axis=0)
    small7 = _knn_small_call(p8cat, ptcat)

    out = []
    for lvl, (v_real, off, pos) in enumerate(zip(_SIZES, _OFFS, vlist)):
        if lvl < 4:
            nbr7 = small7[lvl * 768: lvl * 768 + v_real]
        else:
            p8 = jnp.pad(pos, ((0, 2688 - v_real), (0, 5)))
            pT = jnp.pad(pos.T, ((0, 5), (0, 2688 - v_real)))
            nbr7 = _knn_call(p8, pT, v_real, 2688, 384)
        nbr = nbr7[:v_real, 1:7]
        inten_l = inten_all[off:off + v_real].reshape(v_real, 1)
        feats = jnp.concatenate([inten_l, pos], axis=1)
        src = jnp.broadcast_to(
            jnp.arange(v_real, dtype=jnp.int32)[:, None], (v_real, 6))
        e = jnp.stack([src.reshape(-1), nbr.reshape(-1)], axis=0)
        e = jnp.concatenate([e, e[::-1, :]], axis=1)
        out.extend([pos, feats, e])
    return tuple(out)
